# baseline (speedup 1.0000x reference)
import jax
import jax.numpy as jnp
from jax import lax
from jax.experimental import pallas as pl
from jax.experimental.pallas import tpu as pltpu
from jax.experimental.pallas import tpu_sc as plsc

BATCH = 4096
SEQ = 200
EMBED = 64
VOCAB = 100000
LANES = 16

NUM_CORES = 2
NUM_SUBCORES = 16
NUM_WORKERS = NUM_CORES * NUM_SUBCORES
COLS_PER_WORKER = BATCH // NUM_WORKERS
KCH = COLS_PER_WORKER // LANES

NB = 2


def _body(xT_hbm, tt2_hbm, posT_hbm, out_hbm, xslab, hslab, posT_v, pos_v,
          gbuf, stage, gsem, ssem):
    wid = lax.axis_index("s") * NUM_CORES + lax.axis_index("c")
    cbase = wid * COLS_PER_WORKER

    pltpu.sync_copy(xT_hbm.at[:, pl.ds(cbase, COLS_PER_WORKER)], xslab)
    pltpu.sync_copy(posT_hbm, posT_v)

    @pl.loop(0, SEQ)
    def _pre(s):
        for k in range(KCH):
            sl = pl.ds(k * LANES, LANES)
            v = xslab[s, sl]
            hslab[s, sl] = lax.shift_left(lax.bitwise_and(v, 1), 6)
            xslab[s, sl] = lax.shift_right_logical(v, 1)

    iota = lax.iota(jnp.int32, LANES)
    zero = lax.bitwise_and(iota, 0)

    @pl.loop(0, SEQ)
    def _pt(s):
        sv = jnp.full((LANES,), s, jnp.int32)
        cols = [plsc.load_gather(posT_v, [iota + cq * LANES, sv])
                for cq in range(EMBED // LANES)]
        for cq in range(EMBED // LANES):
            pos_v[pl.ds(s * EMBED + cq * LANES, LANES)] = cols[cq]

    def start_gather(s, g):
        pltpu.async_copy(tt2_hbm.at[xslab.at[s]], gbuf.at[g], gsem.at[g])

    def wait_gather(s, g):
        pltpu.make_async_copy(tt2_hbm.at[xslab.at[s]], gbuf.at[g],
                              gsem.at[g]).wait()

    def drain_store(g):
        pltpu.make_async_copy(stage.at[g],
                              out_hbm.at[0, :, pl.ds(cbase, COLS_PER_WORKER)],
                              ssem.at[g]).wait()

    start_gather(0, 0)

    @pl.loop(0, SEQ, step=NB)
    def _pos(k0):
        for b in range(NB):
            s = k0 + b

            @pl.when(s + 1 < SEQ)
            def _prefetch():
                start_gather(s + 1, (b + 1) % NB)

            wait_gather(s, b)

            @pl.when(s >= NB)
            def _drain():
                drain_store(b)

            base = [hslab[s, pl.ds(k * LANES, LANES)]
                    + lax.shift_left(iota + k * LANES, 7) for k in range(KCH)]

            @pl.loop(0, EMBED // LANES)
            def _colq(q):
                c0 = q * LANES
                for d in range(LANES):
                    dc = lax.bitwise_and(iota + d, LANES - 1) + c0
                    p = plsc.load_gather(pos_v, [dc + s * EMBED])
                    ti = lax.shift_left(dc, 7) + iota
                    vals = [plsc.load_gather(gbuf.at[b], [zero, base[k] + dc])
                            for k in range(KCH)]
                    for k in range(KCH):
                        plsc.store_scatter(stage.at[b],
                                           [zero, ti + k * LANES], vals[k] + p)

            pltpu.async_copy(stage.at[b],
                             out_hbm.at[s, :, pl.ds(cbase, COLS_PER_WORKER)],
                             ssem.at[b])

    for b in range(NB):
        drain_store(b)


def kernel(x, token_table, pos_table):
    x = x.astype(jnp.int32)
    xT = x.T
    tt2 = jnp.reshape(token_table, (VOCAB // 2, 2 * EMBED))
    posT = pos_table.T
    mesh = plsc.VectorSubcoreMesh(
        core_axis_name="c", subcore_axis_name="s",
        num_cores=NUM_CORES, num_subcores=NUM_SUBCORES,
    )
    run = pl.kernel(
        _body,
        out_type=jax.ShapeDtypeStruct((SEQ, EMBED, BATCH), jnp.float32),
        mesh=mesh,
        compiler_params=pltpu.CompilerParams(use_tc_tiling_on_sc=True,
                                             needs_layout_passes=False),
        scratch_types=[
            pltpu.VMEM((SEQ, COLS_PER_WORKER), jnp.int32),
            pltpu.VMEM((SEQ, COLS_PER_WORKER), jnp.int32),
            pltpu.VMEM((EMBED, SEQ), jnp.float32),
            pltpu.VMEM((SEQ * EMBED,), jnp.float32),
            pltpu.VMEM((NB, COLS_PER_WORKER, 2 * EMBED), jnp.float32),
            pltpu.VMEM((NB, EMBED, COLS_PER_WORKER), jnp.float32),
            pltpu.SemaphoreType.DMA((NB,)),
            pltpu.SemaphoreType.DMA((NB,)),
        ],
    )
    o = run(xT, tt2, posT)
    return jnp.transpose(o, (2, 0, 1))


# speedup vs baseline: 1.0359x; 1.0359x over previous
import jax
import jax.numpy as jnp
from jax import lax
from jax.experimental import pallas as pl
from jax.experimental.pallas import tpu as pltpu
from jax.experimental.pallas import tpu_sc as plsc

BATCH = 4096
SEQ = 200
EMBED = 64
VOCAB = 100000
LANES = 16

NUM_CORES = 2
NUM_SUBCORES = 16
NUM_WORKERS = NUM_CORES * NUM_SUBCORES
COLS_PER_WORKER = BATCH // NUM_WORKERS
KCH = COLS_PER_WORKER // LANES

NB = 2


def _body(xT_hbm, tt2_hbm, pos_hbm, out_hbm, xslab, hslab, pos_v, gbuf, stage,
          gsem, ssem):
    wid = lax.axis_index("s") * NUM_CORES + lax.axis_index("c")
    cbase = wid * COLS_PER_WORKER

    pltpu.sync_copy(xT_hbm.at[:, pl.ds(cbase, COLS_PER_WORKER)], xslab)
    pltpu.sync_copy(pos_hbm, pos_v)

    @pl.loop(0, SEQ)
    def _pre(s):
        for k in range(KCH):
            sl = pl.ds(k * LANES, LANES)
            v = xslab[s, sl]
            hslab[s, sl] = lax.shift_left(lax.bitwise_and(v, 1), 6)
            xslab[s, sl] = lax.shift_right_logical(v, 1)

    iota = lax.iota(jnp.int32, LANES)
    zero = lax.bitwise_and(iota, 0)

    def start_gather(s, g):
        pltpu.async_copy(tt2_hbm.at[xslab.at[s]], gbuf.at[g], gsem.at[g])

    def wait_gather(s, g):
        pltpu.make_async_copy(tt2_hbm.at[xslab.at[s]], gbuf.at[g],
                              gsem.at[g]).wait()

    def drain_store(g):
        pltpu.make_async_copy(stage.at[g],
                              out_hbm.at[0, :, pl.ds(cbase, COLS_PER_WORKER)],
                              ssem.at[g]).wait()

    start_gather(0, 0)

    @pl.loop(0, SEQ, step=NB)
    def _pos(k0):
        for b in range(NB):
            s = k0 + b

            @pl.when(s + 1 < SEQ)
            def _prefetch():
                start_gather(s + 1, (b + 1) % NB)

            wait_gather(s, b)

            @pl.when(s >= NB)
            def _drain():
                drain_store(b)

            base = [hslab[s, pl.ds(k * LANES, LANES)]
                    + lax.shift_left(iota + k * LANES, 7) for k in range(KCH)]

            @pl.loop(0, EMBED // LANES)
            def _colq(q):
                c0 = q * LANES
                for d in range(LANES):
                    dc = lax.bitwise_and(iota + d, LANES - 1) + c0
                    p = plsc.load_gather(pos_v, [dc + s * EMBED])
                    ti = lax.shift_left(dc, 7) + iota
                    vals = [plsc.load_gather(gbuf.at[b], [zero, base[k] + dc])
                            for k in range(KCH)]
                    for k in range(KCH):
                        plsc.store_scatter(stage.at[b],
                                           [zero, ti + k * LANES], vals[k] + p)

            pltpu.async_copy(stage.at[b],
                             out_hbm.at[s, :, pl.ds(cbase, COLS_PER_WORKER)],
                             ssem.at[b])

    for b in range(NB):
        drain_store(b)


def kernel(x, token_table, pos_table):
    x = x.astype(jnp.int32)
    xT = x.T
    tt2 = jnp.reshape(token_table, (VOCAB // 2, 2 * EMBED))
    posf = jnp.reshape(pos_table, (-1,))
    mesh = plsc.VectorSubcoreMesh(
        core_axis_name="c", subcore_axis_name="s",
        num_cores=NUM_CORES, num_subcores=NUM_SUBCORES,
    )
    run = pl.kernel(
        _body,
        out_type=jax.ShapeDtypeStruct((SEQ, EMBED, BATCH), jnp.float32),
        mesh=mesh,
        compiler_params=pltpu.CompilerParams(use_tc_tiling_on_sc=True,
                                             needs_layout_passes=False),
        scratch_types=[
            pltpu.VMEM((SEQ, COLS_PER_WORKER), jnp.int32),
            pltpu.VMEM((SEQ, COLS_PER_WORKER), jnp.int32),
            pltpu.VMEM((SEQ * EMBED,), jnp.float32),
            pltpu.VMEM((NB, COLS_PER_WORKER, 2 * EMBED), jnp.float32),
            pltpu.VMEM((NB, EMBED, COLS_PER_WORKER), jnp.float32),
            pltpu.SemaphoreType.DMA((NB,)),
            pltpu.SemaphoreType.DMA((NB,)),
        ],
    )
    o = run(xT, tt2, posf)
    return jnp.transpose(o, (2, 0, 1))
